# Initial kernel scaffold; baseline (speedup 1.0000x reference)
import jax
import jax.numpy as jnp
from jax.experimental import pallas as pl


def kernel(x, router_w, router_b, expert_w, expert_b):
    raise NotImplementedError("write your pallas kernel here")


# speedup vs baseline: 3.6587x; 3.6587x over previous
import jax
import jax.numpy as jnp
from jax.experimental import pallas as pl
from jax.experimental.pallas import tpu as pltpu

N_TOKENS = 4096
D_MODEL = 1024
NUM_EXPERTS = 8
DT = 4
D_TILE = D_MODEL // DT
TB = 512


def _router_body(x_ref, rw_ref, rb_ref, i1_ref, i2_ref, w1_ref, w2_ref):
    logits = jax.lax.dot_general(
        x_ref[...].astype(jnp.bfloat16), rw_ref[...].astype(jnp.bfloat16),
        (((1,), (1,)), ((), ())),
        preferred_element_type=jnp.float32,
    ) + rb_ref[...]
    m = jnp.max(logits, axis=-1, keepdims=True)
    eu = jnp.exp(logits - m)
    p = eu / jnp.sum(eu, axis=-1, keepdims=True)
    iota = jax.lax.broadcasted_iota(jnp.int32, p.shape, 1)
    p1 = jnp.max(p, axis=-1, keepdims=True)
    i1 = jnp.argmax(p, axis=-1)[:, None].astype(jnp.int32)
    pm = jnp.where(iota == i1, -1.0, p)
    p2 = jnp.max(pm, axis=-1, keepdims=True)
    i2 = jnp.argmax(pm, axis=-1)[:, None].astype(jnp.int32)
    s = p1 + p2
    i1_ref[...] = i1
    i2_ref[...] = i2
    w1_ref[...] = p1 / s
    w2_ref[...] = p2 / s


def _expert_body(xbf_ref, w_ref, b_ref, i1_ref, i2_ref, w1_ref, w2_ref,
                 out_ref):
    e = pl.program_id(1)
    pre = jax.lax.dot_general(
        xbf_ref[...], w_ref[0], (((1,), (1,)), ((), ())),
        preferred_element_type=jnp.float32,
    ) + b_ref[0, 0][None, :]
    act = 0.5 * pre * (1.0 + jax.lax.erf(pre * 0.7071067811865476))
    gate = (jnp.where(i1_ref[...] == e, w1_ref[...], 0.0)
            + jnp.where(i2_ref[...] == e, w2_ref[...], 0.0))
    contrib = gate * act

    @pl.when(e == 0)
    def _init():
        out_ref[...] = contrib

    @pl.when(e > 0)
    def _acc():
        out_ref[...] += contrib


@jax.jit
def kernel(x, router_w, router_b, expert_w, expert_b):
    i1, i2, w1, w2 = pl.pallas_call(
        _router_body,
        grid=(N_TOKENS // TB,),
        in_specs=[
            pl.BlockSpec((TB, D_MODEL), lambda t: (t, 0)),
            pl.BlockSpec((NUM_EXPERTS, D_MODEL), lambda t: (0, 0)),
            pl.BlockSpec((1, NUM_EXPERTS), lambda t: (0, 0)),
        ],
        out_specs=[
            pl.BlockSpec((TB, 1), lambda t: (t, 0)),
            pl.BlockSpec((TB, 1), lambda t: (t, 0)),
            pl.BlockSpec((TB, 1), lambda t: (t, 0)),
            pl.BlockSpec((TB, 1), lambda t: (t, 0)),
        ],
        out_shape=[
            jax.ShapeDtypeStruct((N_TOKENS, 1), jnp.int32),
            jax.ShapeDtypeStruct((N_TOKENS, 1), jnp.int32),
            jax.ShapeDtypeStruct((N_TOKENS, 1), jnp.float32),
            jax.ShapeDtypeStruct((N_TOKENS, 1), jnp.float32),
        ],
    )(x, router_w, router_b.reshape(1, NUM_EXPERTS))

    return pl.pallas_call(
        _expert_body,
        grid=(DT, NUM_EXPERTS),
        in_specs=[
            pl.BlockSpec((N_TOKENS, D_MODEL), lambda dt, e: (0, 0)),
            pl.BlockSpec((1, D_TILE, D_MODEL), lambda dt, e: (e, dt, 0)),
            pl.BlockSpec((1, 1, D_TILE), lambda dt, e: (e * DT + dt, 0, 0)),
            pl.BlockSpec((N_TOKENS, 1), lambda dt, e: (0, 0)),
            pl.BlockSpec((N_TOKENS, 1), lambda dt, e: (0, 0)),
            pl.BlockSpec((N_TOKENS, 1), lambda dt, e: (0, 0)),
            pl.BlockSpec((N_TOKENS, 1), lambda dt, e: (0, 0)),
        ],
        out_specs=pl.BlockSpec((N_TOKENS, D_TILE), lambda dt, e: (0, dt)),
        out_shape=jax.ShapeDtypeStruct((N_TOKENS, D_MODEL), jnp.float32),
        compiler_params=pltpu.CompilerParams(
            dimension_semantics=("arbitrary", "arbitrary"),
        ),
    )(x.astype(jnp.bfloat16), expert_w.astype(jnp.bfloat16),
      expert_b.reshape(NUM_EXPERTS * DT, 1, D_TILE), i1, i2, w1, w2)
